# baseline (speedup 1.0000x reference)
import functools

import jax
import jax.numpy as jnp
from jax import lax
from jax.experimental import pallas as pl
from jax.experimental.pallas import tpu as pltpu
from jax.experimental.pallas import tpu_sc as plsc

_LANES = 16


def _tc_ssq_and_rowvals(xt, tgt3, B, C, grid):
    grid_c, grid_b = grid
    CB = C // grid_c
    BP = B // grid_b

    def body(x_ref, t_ref, o_ref, g_ref):
        j = pl.program_id(0)
        i = pl.program_id(1)

        @pl.when(jnp.logical_and(i == 0, j == 0))
        def _init_ssq():
            o_ref[0, 0] = 0.0

        @pl.when(i == 0)
        def _init_g():
            g_ref[0, 0, :] = jnp.zeros((BP,), jnp.float32)

        xb = x_ref[...]
        o_ref[0, 0] += jnp.sum(xb * xb)

        t = t_ref[0, 0, :]
        cls = lax.broadcasted_iota(jnp.int32, (CB, BP), 0) + i * CB
        picked = jnp.where(cls == t[None, :], xb, 0.0)
        g_ref[0, 0, :] += jnp.sum(picked, axis=0)

    return pl.pallas_call(
        body,
        grid=(grid_b, grid_c),
        in_specs=[
            pl.BlockSpec((CB, BP), lambda j, i: (i, j)),
            pl.BlockSpec((1, 1, BP), lambda j, i: (0, 0, j)),
        ],
        out_specs=[
            pl.BlockSpec(memory_space=pltpu.SMEM),
            pl.BlockSpec((1, 1, BP), lambda j, i: (0, 0, j)),
        ],
        out_shape=[
            jax.ShapeDtypeStruct((1, 1), jnp.float32),
            jax.ShapeDtypeStruct((1, 1, B), jnp.float32),
        ],
    )(xt, tgt3)


def _sc_finalize(gathered, ssq16, B, C):
    NC = 2
    NCHUNK = B // _LANES
    inv_n = 1.0 / (B * C)

    mesh = plsc.VectorSubcoreMesh(core_axis_name="c", subcore_axis_name="s")

    @functools.partial(
        pl.kernel,
        out_type=jax.ShapeDtypeStruct((_LANES,), jnp.float32),
        mesh=mesh,
        scratch_types=[
            pltpu.VMEM((B,), jnp.float32),
            pltpu.VMEM((_LANES,), jnp.float32),
            pltpu.VMEM((_LANES,), jnp.float32),
        ],
    )
    def sc_kernel(g_hbm, s_hbm, out_hbm, g_v, s_v, res_v):
        wid = lax.axis_index("s") * NC + lax.axis_index("c")

        @pl.when(wid == 0)
        def _work():
            pltpu.sync_copy(g_hbm, g_v)
            pltpu.sync_copy(s_hbm, s_v)

            UNROLL = 8

            def chunk(j, accs):
                base = j * UNROLL * _LANES
                return tuple(
                    accs[u] + g_v[pl.ds(base + u * _LANES, _LANES)]
                    for u in range(UNROLL)
                )

            zeros = jnp.zeros((_LANES,), jnp.float32)
            accs = lax.fori_loop(0, NCHUNK // UNROLL, chunk,
                                 (zeros,) * UNROLL)
            acc = accs[0]
            for u in range(1, UNROLL):
                acc = acc + accs[u]
            lane = lax.iota(jnp.int32, _LANES)
            for sh in (8, 4, 2, 1):
                acc = acc + acc[lax.bitwise_and(lane + sh, _LANES - 1)]
            res_v[...] = (s_v[...] - 2.0 * acc + float(B)) * inv_n
            pltpu.sync_copy(res_v, out_hbm)

    return sc_kernel(gathered, ssq16)


def kernel(input_score, target):
    B, C = input_score.shape
    GRID = (1, 16)
    xt = input_score.T
    tgt3 = target.reshape(1, 1, B).astype(jnp.int32)
    ssq, gathered3 = _tc_ssq_and_rowvals(xt, tgt3, B, C, GRID)
    gathered = gathered3.reshape(B)
    ssq16 = jnp.broadcast_to(ssq.reshape(1), (_LANES,))
    out = _sc_finalize(gathered, ssq16, B, C)
    return out[0]


# speedup vs baseline: 1.0514x; 1.0514x over previous
import functools

import jax
import jax.numpy as jnp
from jax import lax
from jax.experimental import pallas as pl
from jax.experimental.pallas import tpu as pltpu
from jax.experimental.pallas import tpu_sc as plsc

_LANES = 16


def _tc_ssq_and_rowvals(xt, tgt3, B, C, grid):
    grid_c, grid_b = grid
    CB = C // grid_c
    BP = B // grid_b

    def body(x_ref, t_ref, o_ref, g_ref):
        j = pl.program_id(0)
        i = pl.program_id(1)

        @pl.when(jnp.logical_and(i == 0, j == 0))
        def _init_ssq():
            o_ref[0, 0] = 0.0

        @pl.when(i == 0)
        def _init_g():
            g_ref[0, 0, :] = jnp.zeros((BP,), jnp.float32)

        xb = x_ref[...]
        o_ref[0, 0] += jnp.sum(xb * xb)

        t = t_ref[0, 0, :]
        cls = lax.broadcasted_iota(jnp.int32, (CB, BP), 0) + i * CB
        picked = jnp.where(cls == t[None, :], xb, 0.0)
        g_ref[0, 0, :] += jnp.sum(picked, axis=0)

    return pl.pallas_call(
        body,
        grid=(grid_b, grid_c),
        in_specs=[
            pl.BlockSpec((CB, BP), lambda j, i: (i, j)),
            pl.BlockSpec((1, 1, BP), lambda j, i: (0, 0, j)),
        ],
        out_specs=[
            pl.BlockSpec(memory_space=pltpu.SMEM),
            pl.BlockSpec((1, 1, BP), lambda j, i: (0, 0, j)),
        ],
        out_shape=[
            jax.ShapeDtypeStruct((1, 1), jnp.float32),
            jax.ShapeDtypeStruct((1, 1, B), jnp.float32),
        ],
    )(xt, tgt3)


def _sc_finalize(gathered, ssq16, B, C):
    NC = 2
    NCHUNK = B // _LANES
    inv_n = 1.0 / (B * C)

    mesh = plsc.VectorSubcoreMesh(core_axis_name="c", subcore_axis_name="s")

    @functools.partial(
        pl.kernel,
        out_type=jax.ShapeDtypeStruct((_LANES,), jnp.float32),
        mesh=mesh,
        scratch_types=[
            pltpu.VMEM((B,), jnp.float32),
            pltpu.VMEM((_LANES,), jnp.float32),
            pltpu.VMEM((_LANES,), jnp.float32),
        ],
    )
    def sc_kernel(g_hbm, s_hbm, out_hbm, g_v, s_v, res_v):
        wid = lax.axis_index("s") * NC + lax.axis_index("c")

        @pl.when(wid == 0)
        def _work():
            pltpu.sync_copy(g_hbm, g_v)
            pltpu.sync_copy(s_hbm, s_v)

            UNROLL = 8

            def chunk(j, accs):
                base = j * UNROLL * _LANES
                return tuple(
                    accs[u] + g_v[pl.ds(base + u * _LANES, _LANES)]
                    for u in range(UNROLL)
                )

            zeros = jnp.zeros((_LANES,), jnp.float32)
            accs = lax.fori_loop(0, NCHUNK // UNROLL, chunk,
                                 (zeros,) * UNROLL)
            acc = accs[0]
            for u in range(1, UNROLL):
                acc = acc + accs[u]
            lane = lax.iota(jnp.int32, _LANES)
            for sh in (8, 4, 2, 1):
                acc = acc + acc[lax.bitwise_and(lane + sh, _LANES - 1)]
            res_v[...] = (s_v[...] - 2.0 * acc + float(B)) * inv_n
            pltpu.sync_copy(res_v, out_hbm)

    return sc_kernel(gathered, ssq16)


def kernel(input_score, target):
    B, C = input_score.shape
    GRID = (1, 8)
    xt = input_score.T
    tgt3 = target.reshape(1, 1, B).astype(jnp.int32)
    ssq, gathered3 = _tc_ssq_and_rowvals(xt, tgt3, B, C, GRID)
    gathered = gathered3.reshape(B)
    ssq16 = jnp.broadcast_to(ssq.reshape(1), (_LANES,))
    out = _sc_finalize(gathered, ssq16, B, C)
    return out[0]


# speedup vs baseline: 1.1916x; 1.1334x over previous
import functools

import jax
import jax.numpy as jnp
from jax import lax
from jax.experimental import pallas as pl
from jax.experimental.pallas import tpu as pltpu
from jax.experimental.pallas import tpu_sc as plsc

_LANES = 16


def _tc_ssq_and_rowvals(xt, tgt3, B, C, grid):
    grid_b = grid
    NSPLIT = 5
    CH = C // NSPLIT
    BP = B // grid_b

    def body(*refs):
        x_refs = refs[:NSPLIT]
        t_ref, o_ref, g_ref = refs[NSPLIT:]
        j = pl.program_id(0)

        @pl.when(j == 0)
        def _init_ssq():
            o_ref[0, 0] = 0.0

        t = t_ref[0, 0, :]
        acc = jnp.zeros((BP,), jnp.float32)
        ssq = 0.0
        for s, x_ref in enumerate(x_refs):
            xb = x_ref[...]
            ssq += jnp.sum(xb * xb)
            cls = lax.broadcasted_iota(jnp.int32, (CH, BP), 0) + s * CH
            acc = acc + jnp.sum(jnp.where(cls == t[None, :], xb, 0.0),
                                axis=0)
        o_ref[0, 0] += ssq
        g_ref[0, 0, :] = acc

    def _mk_spec(s):
        return pl.BlockSpec((CH, BP), lambda j, s=s: (s, j))

    return pl.pallas_call(
        body,
        grid=(grid_b,),
        in_specs=[_mk_spec(s) for s in range(NSPLIT)] + [
            pl.BlockSpec((1, 1, BP), lambda j: (0, 0, j)),
        ],
        out_specs=[
            pl.BlockSpec(memory_space=pltpu.SMEM),
            pl.BlockSpec((1, 1, BP), lambda j: (0, 0, j)),
        ],
        out_shape=[
            jax.ShapeDtypeStruct((1, 1), jnp.float32),
            jax.ShapeDtypeStruct((1, 1, B), jnp.float32),
        ],
    )(*([xt] * NSPLIT + [tgt3]))


def _sc_finalize(gathered, ssq16, B, C):
    NC = 2
    NCHUNK = B // _LANES
    inv_n = 1.0 / (B * C)

    mesh = plsc.VectorSubcoreMesh(core_axis_name="c", subcore_axis_name="s")

    @functools.partial(
        pl.kernel,
        out_type=jax.ShapeDtypeStruct((_LANES,), jnp.float32),
        mesh=mesh,
        scratch_types=[
            pltpu.VMEM((B,), jnp.float32),
            pltpu.VMEM((_LANES,), jnp.float32),
            pltpu.VMEM((_LANES,), jnp.float32),
        ],
    )
    def sc_kernel(g_hbm, s_hbm, out_hbm, g_v, s_v, res_v):
        wid = lax.axis_index("s") * NC + lax.axis_index("c")

        @pl.when(wid == 0)
        def _work():
            pltpu.sync_copy(g_hbm, g_v)
            pltpu.sync_copy(s_hbm, s_v)

            UNROLL = 8

            def chunk(j, accs):
                base = j * UNROLL * _LANES
                return tuple(
                    accs[u] + g_v[pl.ds(base + u * _LANES, _LANES)]
                    for u in range(UNROLL)
                )

            zeros = jnp.zeros((_LANES,), jnp.float32)
            accs = lax.fori_loop(0, NCHUNK // UNROLL, chunk,
                                 (zeros,) * UNROLL)
            acc = accs[0]
            for u in range(1, UNROLL):
                acc = acc + accs[u]
            lane = lax.iota(jnp.int32, _LANES)
            for sh in (8, 4, 2, 1):
                acc = acc + acc[lax.bitwise_and(lane + sh, _LANES - 1)]
            res_v[...] = (s_v[...] - 2.0 * acc + float(B)) * inv_n
            pltpu.sync_copy(res_v, out_hbm)

    return sc_kernel(gathered, ssq16)


def kernel(input_score, target):
    B, C = input_score.shape
    GRID = 8
    xt = input_score.T
    tgt3 = target.reshape(1, 1, B).astype(jnp.int32)
    ssq, gathered3 = _tc_ssq_and_rowvals(xt, tgt3, B, C, GRID)
    gathered = gathered3.reshape(B)
    ssq16 = jnp.broadcast_to(ssq.reshape(1), (_LANES,))
    out = _sc_finalize(gathered, ssq16, B, C)
    return out[0]
